# Initial kernel scaffold; baseline (speedup 1.0000x reference)
import jax
import jax.numpy as jnp
from jax.experimental import pallas as pl


def kernel(sequence, node_table):
    raise NotImplementedError("write your pallas kernel here")


# speedup vs baseline: 32.2204x; 32.2204x over previous
import functools

import jax
import jax.numpy as jnp
from jax import lax
from jax.experimental import pallas as pl
from jax.experimental.pallas import tpu as pltpu
from jax.experimental.pallas import tpu_sc as plsc

VOCAB = 100000
ROW = 256
NFLAT = 4096 * 200
NC, NS, NW, L = 2, 16, 32, 16
VPAD = 102400
SCAN_W = VPAD // NW
SEQ_W = NFLAT // NW
CHUNK = 128
K1_CHUNKS = SEQ_W // CHUNK
K3_CHUNKS = 25

_mesh = plsc.VectorSubcoreMesh(core_axis_name="c", subcore_axis_name="s")
_sc_params = pltpu.CompilerParams(needs_layout_passes=False)


def _wid():
    return lax.axis_index("c") * NS + lax.axis_index("s")


def _k1_body(seq_hbm, present_hbm, idx_v, ones_v, z_v, bitmap_sp):
    c = lax.axis_index("c")
    s = lax.axis_index("s")
    wid = c * NS + s

    def zero_z(i, _):
        z_v[pl.ds(i * L, L)] = jnp.zeros((L,), jnp.int32)
        return 0

    lax.fori_loop(0, (VPAD // NS) // L, zero_z, 0)
    pltpu.sync_copy(z_v, bitmap_sp.at[pl.ds(s * (VPAD // NS), VPAD // NS)])
    plsc.subcore_barrier()

    def fill_ones(i, _):
        ones_v[pl.ds(i * L, L)] = jnp.ones((L,), jnp.int32)
        return 0

    lax.fori_loop(0, CHUNK // L, fill_ones, 0)

    def scatter(j, _):
        pltpu.sync_copy(seq_hbm.at[pl.ds(wid * SEQ_W + j * CHUNK, CHUNK)], idx_v)
        pltpu.sync_copy(ones_v, bitmap_sp.at[idx_v])
        return 0

    lax.fori_loop(0, K1_CHUNKS, scatter, 0)
    plsc.subcore_barrier()
    sl = pl.ds(s * (VPAD // NS), VPAD // NS)
    pltpu.sync_copy(bitmap_sp.at[sl], present_hbm.at[c, sl])


_k1 = functools.partial(
    pl.kernel,
    out_type=jax.ShapeDtypeStruct((NC, VPAD), jnp.int32),
    mesh=_mesh,
    compiler_params=_sc_params,
    scratch_types=[
        pltpu.VMEM((CHUNK,), jnp.int32),
        pltpu.VMEM((CHUNK,), jnp.int32),
        pltpu.VMEM((VPAD // NS,), jnp.int32),
        pltpu.VMEM_SHARED((VPAD,), jnp.int32),
    ],
)(_k1_body)


def _k2a_body(present_hbm, wsums_hbm, p0_v, p1_v, s_v):
    wid = _wid()
    sl = pl.ds(wid * SCAN_W, SCAN_W)
    pltpu.sync_copy(present_hbm.at[0, sl], p0_v)
    pltpu.sync_copy(present_hbm.at[1, sl], p1_v)

    def body(k, acc):
        p = p0_v[pl.ds(k * L, L)] + p1_v[pl.ds(k * L, L)]
        return acc + jnp.where(p > 0, 1, 0).astype(jnp.int32)

    acc = lax.fori_loop(0, SCAN_W // L, body, jnp.zeros((L,), jnp.int32))
    total = jnp.sum(acc)
    s_v[...] = jnp.full((L,), total, jnp.int32)
    pltpu.sync_copy(s_v, wsums_hbm.at[wid])


_k2a = functools.partial(
    pl.kernel,
    out_type=jax.ShapeDtypeStruct((NW, L), jnp.int32),
    mesh=_mesh,
    compiler_params=_sc_params,
    scratch_types=[
        pltpu.VMEM((SCAN_W,), jnp.int32),
        pltpu.VMEM((SCAN_W,), jnp.int32),
        pltpu.VMEM((L,), jnp.int32),
    ],
)(_k2a_body)


def _k2b_body(present_hbm, wsums_hbm, pos_hbm, nfill_hbm, p0_v, p1_v, pos_v, w_v, nf_v):
    wid = _wid()
    pltpu.sync_copy(wsums_hbm, w_v)

    def sums(r, carry):
        total, offset = carry
        sr = jnp.max(w_v[r])
        return total + sr, offset + jnp.where(r < wid, sr, jnp.int32(0))

    total, offset = lax.fori_loop(0, NW, sums, (jnp.int32(0), jnp.int32(0)))
    n_fill = jnp.int32(VOCAB) - total

    sl = pl.ds(wid * SCAN_W, SCAN_W)
    pltpu.sync_copy(present_hbm.at[0, sl], p0_v)
    pltpu.sync_copy(present_hbm.at[1, sl], p1_v)

    def scan(k, carry):
        p = jnp.where(p0_v[pl.ds(k * L, L)] + p1_v[pl.ds(k * L, L)] > 0, 1, 0)
        p = p.astype(jnp.int32)
        incl = plsc.cumsum(p)
        excl = incl - p
        pos_v[pl.ds(k * L, L)] = jnp.where(p > 0, n_fill + carry + excl, 0)
        return carry + jnp.sum(p)

    lax.fori_loop(0, SCAN_W // L, scan, offset)
    pltpu.sync_copy(pos_v, pos_hbm.at[sl])

    @pl.when(wid == 0)
    def _():
        nf_v[...] = jnp.full((L,), n_fill, jnp.int32)
        pltpu.sync_copy(nf_v, nfill_hbm)


_k2b = functools.partial(
    pl.kernel,
    out_type=(
        jax.ShapeDtypeStruct((VPAD,), jnp.int32),
        jax.ShapeDtypeStruct((L,), jnp.int32),
    ),
    mesh=_mesh,
    compiler_params=_sc_params,
    scratch_types=[
        pltpu.VMEM((SCAN_W,), jnp.int32),
        pltpu.VMEM((SCAN_W,), jnp.int32),
        pltpu.VMEM((SCAN_W,), jnp.int32),
        pltpu.VMEM((NW, L), jnp.int32),
        pltpu.VMEM((L,), jnp.int32),
    ],
)(_k2b_body)


def _k3_body(table_hbm, pos_hbm, out_hbm, rows_v, idx_v, sem):
    wid = _wid()
    b = lax.div(jnp.int32(3125) * wid, jnp.int32(8)) * 8
    e = jnp.where(
        wid == NW - 1,
        jnp.int32(VOCAB),
        lax.div(jnp.int32(3125) * (wid + 1), jnp.int32(8)) * 8,
    )

    def chunk(j, _):
        base = jnp.minimum(b + j * CHUNK, e - CHUNK)
        base = pl.multiple_of(base, 8)
        pltpu.sync_copy(table_hbm.at[pl.ds(base, CHUNK)], rows_v)
        pltpu.sync_copy(pos_hbm.at[pl.ds(base, CHUNK)], idx_v)
        pltpu.async_copy(rows_v, out_hbm.at[idx_v], sem).wait()
        return 0

    lax.fori_loop(0, K3_CHUNKS, chunk, 0)


_k3 = functools.partial(
    pl.kernel,
    out_type=jax.ShapeDtypeStruct((VOCAB, ROW), jnp.float32),
    mesh=_mesh,
    compiler_params=_sc_params,
    scratch_types=[
        pltpu.VMEM((CHUNK, ROW), jnp.float32),
        pltpu.VMEM((CHUNK,), jnp.int32),
        pltpu.SemaphoreType.DMA,
    ],
)(_k3_body)


def _k4_body(nfill_ref, out_in_ref, out_ref, z_v, sem):
    del out_in_ref
    z_v[...] = jnp.zeros_like(z_v)
    n = nfill_ref[0]
    nb = n // 8

    def blk(i, _):
        cp = pltpu.make_async_copy(z_v, out_ref.at[pl.ds(i * 8, 8)], sem)
        cp.start()
        cp.wait()
        return 0

    lax.fori_loop(0, nb, blk, 0)

    def row(i, _):
        cp = pltpu.make_async_copy(
            z_v.at[pl.ds(0, 1)], out_ref.at[pl.ds(nb * 8 + i, 1)], sem
        )
        cp.start()
        cp.wait()
        return 0

    lax.fori_loop(0, n - nb * 8, row, 0)


_k4 = pl.pallas_call(
    _k4_body,
    out_shape=jax.ShapeDtypeStruct((VOCAB, ROW), jnp.float32),
    in_specs=[
        pl.BlockSpec(memory_space=pltpu.SMEM),
        pl.BlockSpec(memory_space=pl.ANY),
    ],
    out_specs=pl.BlockSpec(memory_space=pl.ANY),
    scratch_shapes=[pltpu.VMEM((8, ROW), jnp.float32), pltpu.SemaphoreType.DMA],
    input_output_aliases={1: 0},
)


@jax.jit
def kernel(sequence, node_table):
    seq_flat = sequence.reshape(-1)
    table2 = node_table.reshape(VOCAB, ROW)
    present = _k1(seq_flat)
    wsums = _k2a(present)
    pos, nfill = _k2b(present, wsums)
    out = _k3(table2, pos)
    out = _k4(nfill[:1], out)
    return out.reshape(VOCAB, 8, 32)


# speedup vs baseline: 34.5386x; 1.0719x over previous
import functools

import jax
import jax.numpy as jnp
from jax import lax
from jax.experimental import pallas as pl
from jax.experimental.pallas import tpu as pltpu
from jax.experimental.pallas import tpu_sc as plsc

VOCAB = 100000
ROW = 256
NFLAT = 4096 * 200
NC, NS, NW, L = 2, 16, 32, 16
VPAD = 102400
SCAN_W = VPAD // NW
SEQ_W = NFLAT // NW
CHUNK = 128
K1_CHUNKS = SEQ_W // CHUNK
K3_CHUNKS = 25

_mesh = plsc.VectorSubcoreMesh(core_axis_name="c", subcore_axis_name="s")
_sc_params = pltpu.CompilerParams(needs_layout_passes=False)


def _wid():
    return lax.axis_index("c") * NS + lax.axis_index("s")


def _k1_body(seq_hbm, present_hbm, idx_v, ones_v, z_v, bitmap_sp):
    c = lax.axis_index("c")
    s = lax.axis_index("s")
    wid = c * NS + s

    def zero_z(i, _):
        z_v[pl.ds(i * L, L)] = jnp.zeros((L,), jnp.int32)
        return 0

    lax.fori_loop(0, (VPAD // NS) // L, zero_z, 0)
    pltpu.sync_copy(z_v, bitmap_sp.at[pl.ds(s * (VPAD // NS), VPAD // NS)])
    plsc.subcore_barrier()

    def fill_ones(i, _):
        ones_v[pl.ds(i * L, L)] = jnp.ones((L,), jnp.int32)
        return 0

    lax.fori_loop(0, CHUNK // L, fill_ones, 0)

    def scatter(j, _):
        pltpu.sync_copy(seq_hbm.at[pl.ds(wid * SEQ_W + j * CHUNK, CHUNK)], idx_v)
        pltpu.sync_copy(ones_v, bitmap_sp.at[idx_v])
        return 0

    lax.fori_loop(0, K1_CHUNKS, scatter, 0)
    plsc.subcore_barrier()
    sl = pl.ds(s * (VPAD // NS), VPAD // NS)
    pltpu.sync_copy(bitmap_sp.at[sl], present_hbm.at[c, sl])


_k1 = functools.partial(
    pl.kernel,
    out_type=jax.ShapeDtypeStruct((NC, VPAD), jnp.int32),
    mesh=_mesh,
    compiler_params=_sc_params,
    scratch_types=[
        pltpu.VMEM((CHUNK,), jnp.int32),
        pltpu.VMEM((CHUNK,), jnp.int32),
        pltpu.VMEM((VPAD // NS,), jnp.int32),
        pltpu.VMEM_SHARED((VPAD,), jnp.int32),
    ],
)(_k1_body)


def _k2a_body(present_hbm, wsums_hbm, p0_v, p1_v, s_v):
    wid = _wid()
    sl = pl.ds(wid * SCAN_W, SCAN_W)
    pltpu.sync_copy(present_hbm.at[0, sl], p0_v)
    pltpu.sync_copy(present_hbm.at[1, sl], p1_v)

    def body(k, acc):
        p = p0_v[pl.ds(k * L, L)] + p1_v[pl.ds(k * L, L)]
        return acc + jnp.where(p > 0, 1, 0).astype(jnp.int32)

    acc = lax.fori_loop(0, SCAN_W // L, body, jnp.zeros((L,), jnp.int32))
    total = jnp.sum(acc)
    s_v[...] = jnp.full((L,), total, jnp.int32)
    pltpu.sync_copy(s_v, wsums_hbm.at[wid])


_k2a = functools.partial(
    pl.kernel,
    out_type=jax.ShapeDtypeStruct((NW, L), jnp.int32),
    mesh=_mesh,
    compiler_params=_sc_params,
    scratch_types=[
        pltpu.VMEM((SCAN_W,), jnp.int32),
        pltpu.VMEM((SCAN_W,), jnp.int32),
        pltpu.VMEM((L,), jnp.int32),
    ],
)(_k2a_body)


def _k2b_body(present_hbm, wsums_hbm, pos_hbm, nfill_hbm, p0_v, p1_v, pos_v, w_v, nf_v):
    wid = _wid()
    pltpu.sync_copy(wsums_hbm, w_v)

    def sums(r, carry):
        total, offset = carry
        sr = jnp.max(w_v[r])
        return total + sr, offset + jnp.where(r < wid, sr, jnp.int32(0))

    total, offset = lax.fori_loop(0, NW, sums, (jnp.int32(0), jnp.int32(0)))
    n_fill = jnp.int32(VOCAB) - total

    sl = pl.ds(wid * SCAN_W, SCAN_W)
    pltpu.sync_copy(present_hbm.at[0, sl], p0_v)
    pltpu.sync_copy(present_hbm.at[1, sl], p1_v)

    def scan(k, carry):
        p = jnp.where(p0_v[pl.ds(k * L, L)] + p1_v[pl.ds(k * L, L)] > 0, 1, 0)
        p = p.astype(jnp.int32)
        incl = plsc.cumsum(p)
        excl = incl - p
        pos_v[pl.ds(k * L, L)] = jnp.where(p > 0, n_fill + carry + excl, 0)
        return carry + jnp.sum(p)

    lax.fori_loop(0, SCAN_W // L, scan, offset)
    pltpu.sync_copy(pos_v, pos_hbm.at[sl])

    @pl.when(wid == 0)
    def _():
        nf_v[...] = jnp.full((L,), n_fill, jnp.int32)
        pltpu.sync_copy(nf_v, nfill_hbm)


_k2b = functools.partial(
    pl.kernel,
    out_type=(
        jax.ShapeDtypeStruct((VPAD,), jnp.int32),
        jax.ShapeDtypeStruct((L,), jnp.int32),
    ),
    mesh=_mesh,
    compiler_params=_sc_params,
    scratch_types=[
        pltpu.VMEM((SCAN_W,), jnp.int32),
        pltpu.VMEM((SCAN_W,), jnp.int32),
        pltpu.VMEM((SCAN_W,), jnp.int32),
        pltpu.VMEM((NW, L), jnp.int32),
        pltpu.VMEM((L,), jnp.int32),
    ],
)(_k2b_body)


def _k3_body(table_hbm, pos_hbm, out_hbm, rows0, rows1, idx0, idx1,
             sg0, sg1, si0, si1, ss0, ss1):
    wid = _wid()
    rows = (rows0, rows1)
    idx = (idx0, idx1)
    sg = (sg0, sg1)
    si = (si0, si1)
    ss = (ss0, ss1)
    b = lax.div(jnp.int32(3125) * wid, jnp.int32(8)) * 8
    e = jnp.where(
        wid == NW - 1,
        jnp.int32(VOCAB),
        lax.div(jnp.int32(3125) * (wid + 1), jnp.int32(8)) * 8,
    )

    def start_gather(j, p):
        base = jnp.minimum(b + j * CHUNK, e - CHUNK)
        base = pl.multiple_of(base, 8)
        pltpu.async_copy(table_hbm.at[pl.ds(base, CHUNK)], rows[p], sg[p])
        pltpu.async_copy(pos_hbm.at[pl.ds(base, CHUNK)], idx[p], si[p])

    def wait_gather(p):
        pltpu.make_async_copy(table_hbm.at[pl.ds(0, CHUNK)], rows[p], sg[p]).wait()
        pltpu.make_async_copy(pos_hbm.at[pl.ds(0, CHUNK)], idx[p], si[p]).wait()

    def wait_scatter(p):
        pltpu.make_async_copy(table_hbm.at[pl.ds(0, CHUNK)], rows[p], ss[p]).wait()

    start_gather(0, 0)

    def iter_t(t, _):
        for phase in range(2):
            j = 2 * t + phase

            @pl.when(j < K3_CHUNKS)
            def _():
                @pl.when(j + 1 < K3_CHUNKS)
                def _():
                    @pl.when(j >= 1)
                    def _():
                        wait_scatter(1 - phase)

                    start_gather(j + 1, 1 - phase)

                wait_gather(phase)
                pltpu.async_copy(rows[phase], out_hbm.at[idx[phase]], ss[phase])
        return 0

    lax.fori_loop(0, (K3_CHUNKS + 1) // 2, iter_t, 0)
    wait_scatter(1)
    wait_scatter(0)


_k3 = functools.partial(
    pl.kernel,
    out_type=jax.ShapeDtypeStruct((VOCAB, ROW), jnp.float32),
    mesh=_mesh,
    compiler_params=_sc_params,
    scratch_types=[
        pltpu.VMEM((CHUNK, ROW), jnp.float32),
        pltpu.VMEM((CHUNK, ROW), jnp.float32),
        pltpu.VMEM((CHUNK,), jnp.int32),
        pltpu.VMEM((CHUNK,), jnp.int32),
        pltpu.SemaphoreType.DMA,
        pltpu.SemaphoreType.DMA,
        pltpu.SemaphoreType.DMA,
        pltpu.SemaphoreType.DMA,
        pltpu.SemaphoreType.DMA,
        pltpu.SemaphoreType.DMA,
    ],
)(_k3_body)


def _k4_body(nfill_ref, out_in_ref, out_ref, z_v, sem):
    del out_in_ref
    z_v[...] = jnp.zeros_like(z_v)
    n = nfill_ref[0]
    nb = n // 8

    def blk(i, _):
        cp = pltpu.make_async_copy(z_v, out_ref.at[pl.ds(i * 8, 8)], sem)
        cp.start()
        cp.wait()
        return 0

    lax.fori_loop(0, nb, blk, 0)

    def row(i, _):
        cp = pltpu.make_async_copy(
            z_v.at[pl.ds(0, 1)], out_ref.at[pl.ds(nb * 8 + i, 1)], sem
        )
        cp.start()
        cp.wait()
        return 0

    lax.fori_loop(0, n - nb * 8, row, 0)


_k4 = pl.pallas_call(
    _k4_body,
    out_shape=jax.ShapeDtypeStruct((VOCAB, ROW), jnp.float32),
    in_specs=[
        pl.BlockSpec(memory_space=pltpu.SMEM),
        pl.BlockSpec(memory_space=pl.ANY),
    ],
    out_specs=pl.BlockSpec(memory_space=pl.ANY),
    scratch_shapes=[pltpu.VMEM((8, ROW), jnp.float32), pltpu.SemaphoreType.DMA],
    input_output_aliases={1: 0},
)


@jax.jit
def kernel(sequence, node_table):
    table2 = node_table.reshape(VOCAB, ROW)
    present = _k1(sequence.reshape(-1))
    wsums = _k2a(present)
    pos, nfill = _k2b(present, wsums)
    out = _k3(table2, pos)
    out = _k4(nfill[:1], out)
    return out.reshape(VOCAB, 8, 32)


# speedup vs baseline: 44.4215x; 1.2861x over previous
import functools

import jax
import jax.numpy as jnp
from jax import lax
from jax.experimental import pallas as pl
from jax.experimental.pallas import tpu as pltpu
from jax.experimental.pallas import tpu_sc as plsc

VOCAB = 100000
ROW = 256
NFLAT = 4096 * 200
NC, NS, NW, L = 2, 16, 32, 16
VPAD = 102400
SCAN_W = VPAD // NW
SEQ_W = NFLAT // NW
CHUNK = 128
K1_CHUNKS = SEQ_W // CHUNK
K3_CHUNKS = 25

_mesh = plsc.VectorSubcoreMesh(core_axis_name="c", subcore_axis_name="s")
_sc_params = pltpu.CompilerParams(needs_layout_passes=False)


def _wid():
    return lax.axis_index("c") * NS + lax.axis_index("s")


K1_RING = 8


def _k1_body(seq_hbm, present_hbm, *refs):
    bufs = refs[:K1_RING]
    ones_v = refs[K1_RING]
    z_v = refs[K1_RING + 1]
    bitmap_sp = refs[K1_RING + 2]
    slo = refs[K1_RING + 3:K1_RING + 3 + K1_RING]
    ssc = refs[K1_RING + 3 + K1_RING:]
    c = lax.axis_index("c")
    s = lax.axis_index("s")
    wid = c * NS + s

    def zero_z(i, _):
        z_v[pl.ds(i * L, L)] = jnp.zeros((L,), jnp.int32)
        return 0

    lax.fori_loop(0, (VPAD // NS) // L, zero_z, 0)
    pltpu.sync_copy(z_v, bitmap_sp.at[pl.ds(s * (VPAD // NS), VPAD // NS)])
    plsc.subcore_barrier()

    def fill_ones(i, _):
        ones_v[pl.ds(i * L, L)] = jnp.ones((L,), jnp.int32)
        return 0

    lax.fori_loop(0, CHUNK // L, fill_ones, 0)

    base = wid * SEQ_W

    def start_load(j, k):
        pltpu.async_copy(seq_hbm.at[pl.ds(base + j * CHUNK, CHUNK)], bufs[k], slo[k])

    def wait_load(k):
        pltpu.make_async_copy(seq_hbm.at[pl.ds(0, CHUNK)], bufs[k], slo[k]).wait()

    def drain_sc(k):
        pltpu.make_async_copy(seq_hbm.at[pl.ds(0, CHUNK)], bufs[k], ssc[k]).wait()

    for k in range(K1_RING):
        start_load(k, k)

    def round_(r, _):
        for k in range(K1_RING):
            j = K1_RING * r + k
            wait_load(k)
            pltpu.async_copy(ones_v, bitmap_sp.at[bufs[k]], ssc[k])

            @pl.when(j + K1_RING < K1_CHUNKS)
            def _():
                drain_sc(k)
                start_load(j + K1_RING, k)
        return 0

    lax.fori_loop(0, K1_CHUNKS // K1_RING, round_, 0)
    for k in range(K1_RING):
        drain_sc(k)
    plsc.subcore_barrier()
    sl = pl.ds(s * (VPAD // NS), VPAD // NS)
    pltpu.sync_copy(bitmap_sp.at[sl], present_hbm.at[c, sl])


_k1 = functools.partial(
    pl.kernel,
    out_type=jax.ShapeDtypeStruct((NC, VPAD), jnp.int32),
    mesh=_mesh,
    compiler_params=_sc_params,
    scratch_types=(
        [pltpu.VMEM((CHUNK,), jnp.int32)] * K1_RING
        + [
            pltpu.VMEM((CHUNK,), jnp.int32),
            pltpu.VMEM((VPAD // NS,), jnp.int32),
            pltpu.VMEM_SHARED((VPAD,), jnp.int32),
        ]
        + [pltpu.SemaphoreType.DMA] * (2 * K1_RING)
    ),
)(_k1_body)


def _k2a_body(present_hbm, wsums_hbm, p0_v, p1_v, s_v):
    wid = _wid()
    sl = pl.ds(wid * SCAN_W, SCAN_W)
    pltpu.sync_copy(present_hbm.at[0, sl], p0_v)
    pltpu.sync_copy(present_hbm.at[1, sl], p1_v)

    def body(k, acc):
        p = p0_v[pl.ds(k * L, L)] + p1_v[pl.ds(k * L, L)]
        return acc + jnp.where(p > 0, 1, 0).astype(jnp.int32)

    acc = lax.fori_loop(0, SCAN_W // L, body, jnp.zeros((L,), jnp.int32))
    total = jnp.sum(acc)
    s_v[...] = jnp.full((L,), total, jnp.int32)
    pltpu.sync_copy(s_v, wsums_hbm.at[wid])


_k2a = functools.partial(
    pl.kernel,
    out_type=jax.ShapeDtypeStruct((NW, L), jnp.int32),
    mesh=_mesh,
    compiler_params=_sc_params,
    scratch_types=[
        pltpu.VMEM((SCAN_W,), jnp.int32),
        pltpu.VMEM((SCAN_W,), jnp.int32),
        pltpu.VMEM((L,), jnp.int32),
    ],
)(_k2a_body)


def _k2b_body(present_hbm, wsums_hbm, pos_hbm, nfill_hbm, p0_v, p1_v, pos_v, w_v, nf_v):
    wid = _wid()
    pltpu.sync_copy(wsums_hbm, w_v)

    def sums(r, carry):
        total, offset = carry
        sr = jnp.max(w_v[r])
        return total + sr, offset + jnp.where(r < wid, sr, jnp.int32(0))

    total, offset = lax.fori_loop(0, NW, sums, (jnp.int32(0), jnp.int32(0)))
    n_fill = jnp.int32(VOCAB) - total

    sl = pl.ds(wid * SCAN_W, SCAN_W)
    pltpu.sync_copy(present_hbm.at[0, sl], p0_v)
    pltpu.sync_copy(present_hbm.at[1, sl], p1_v)

    def scan(k, carry):
        p = jnp.where(p0_v[pl.ds(k * L, L)] + p1_v[pl.ds(k * L, L)] > 0, 1, 0)
        p = p.astype(jnp.int32)
        incl = plsc.cumsum(p)
        excl = incl - p
        pos_v[pl.ds(k * L, L)] = jnp.where(p > 0, n_fill + carry + excl, 0)
        return carry + jnp.sum(p)

    lax.fori_loop(0, SCAN_W // L, scan, offset)
    pltpu.sync_copy(pos_v, pos_hbm.at[sl])

    @pl.when(wid == 0)
    def _():
        nf_v[...] = jnp.full((L,), n_fill, jnp.int32)
        pltpu.sync_copy(nf_v, nfill_hbm)


_k2b = functools.partial(
    pl.kernel,
    out_type=(
        jax.ShapeDtypeStruct((VPAD,), jnp.int32),
        jax.ShapeDtypeStruct((L,), jnp.int32),
    ),
    mesh=_mesh,
    compiler_params=_sc_params,
    scratch_types=[
        pltpu.VMEM((SCAN_W,), jnp.int32),
        pltpu.VMEM((SCAN_W,), jnp.int32),
        pltpu.VMEM((SCAN_W,), jnp.int32),
        pltpu.VMEM((NW, L), jnp.int32),
        pltpu.VMEM((L,), jnp.int32),
    ],
)(_k2b_body)


def _k3_body(table_hbm, pos_hbm, out_hbm, rows0, rows1, idx0, idx1,
             sg0, sg1, si0, si1, ss0, ss1):
    wid = _wid()
    rows = (rows0, rows1)
    idx = (idx0, idx1)
    sg = (sg0, sg1)
    si = (si0, si1)
    ss = (ss0, ss1)
    b = lax.div(jnp.int32(3125) * wid, jnp.int32(8)) * 8
    e = jnp.where(
        wid == NW - 1,
        jnp.int32(VOCAB),
        lax.div(jnp.int32(3125) * (wid + 1), jnp.int32(8)) * 8,
    )

    def start_gather(j, p):
        base = jnp.minimum(b + j * CHUNK, e - CHUNK)
        base = pl.multiple_of(base, 8)
        pltpu.async_copy(table_hbm.at[pl.ds(base, CHUNK)], rows[p], sg[p])
        pltpu.async_copy(pos_hbm.at[pl.ds(base, CHUNK)], idx[p], si[p])

    def wait_gather(p):
        pltpu.make_async_copy(table_hbm.at[pl.ds(0, CHUNK)], rows[p], sg[p]).wait()
        pltpu.make_async_copy(pos_hbm.at[pl.ds(0, CHUNK)], idx[p], si[p]).wait()

    def wait_scatter(p):
        pltpu.make_async_copy(table_hbm.at[pl.ds(0, CHUNK)], rows[p], ss[p]).wait()

    start_gather(0, 0)

    def iter_t(t, _):
        for phase in range(2):
            j = 2 * t + phase

            @pl.when(j < K3_CHUNKS)
            def _():
                @pl.when(j + 1 < K3_CHUNKS)
                def _():
                    @pl.when(j >= 1)
                    def _():
                        wait_scatter(1 - phase)

                    start_gather(j + 1, 1 - phase)

                wait_gather(phase)
                pltpu.async_copy(rows[phase], out_hbm.at[idx[phase]], ss[phase])
        return 0

    lax.fori_loop(0, (K3_CHUNKS + 1) // 2, iter_t, 0)
    wait_scatter(1)
    wait_scatter(0)


_k3 = functools.partial(
    pl.kernel,
    out_type=jax.ShapeDtypeStruct((VOCAB, ROW), jnp.float32),
    mesh=_mesh,
    compiler_params=_sc_params,
    scratch_types=[
        pltpu.VMEM((CHUNK, ROW), jnp.float32),
        pltpu.VMEM((CHUNK, ROW), jnp.float32),
        pltpu.VMEM((CHUNK,), jnp.int32),
        pltpu.VMEM((CHUNK,), jnp.int32),
        pltpu.SemaphoreType.DMA,
        pltpu.SemaphoreType.DMA,
        pltpu.SemaphoreType.DMA,
        pltpu.SemaphoreType.DMA,
        pltpu.SemaphoreType.DMA,
        pltpu.SemaphoreType.DMA,
    ],
)(_k3_body)


def _k4_body(nfill_ref, out_in_ref, out_ref, z_v, sem):
    del out_in_ref
    z_v[...] = jnp.zeros_like(z_v)
    n = nfill_ref[0]
    nb = n // 8

    def blk(i, _):
        cp = pltpu.make_async_copy(z_v, out_ref.at[pl.ds(i * 8, 8)], sem)
        cp.start()
        cp.wait()
        return 0

    lax.fori_loop(0, nb, blk, 0)

    def row(i, _):
        cp = pltpu.make_async_copy(
            z_v.at[pl.ds(0, 1)], out_ref.at[pl.ds(nb * 8 + i, 1)], sem
        )
        cp.start()
        cp.wait()
        return 0

    lax.fori_loop(0, n - nb * 8, row, 0)


_k4 = pl.pallas_call(
    _k4_body,
    out_shape=jax.ShapeDtypeStruct((VOCAB, ROW), jnp.float32),
    in_specs=[
        pl.BlockSpec(memory_space=pltpu.SMEM),
        pl.BlockSpec(memory_space=pl.ANY),
    ],
    out_specs=pl.BlockSpec(memory_space=pl.ANY),
    scratch_shapes=[pltpu.VMEM((8, ROW), jnp.float32), pltpu.SemaphoreType.DMA],
    input_output_aliases={1: 0},
)


@jax.jit
def kernel(sequence, node_table):
    table2 = node_table.reshape(VOCAB, ROW)
    present = _k1(sequence.reshape(-1))
    wsums = _k2a(present)
    pos, nfill = _k2b(present, wsums)
    out = _k3(table2, pos)
    out = _k4(nfill[:1], out)
    return out.reshape(VOCAB, 8, 32)


# speedup vs baseline: 45.0574x; 1.0143x over previous
import functools

import jax
import jax.numpy as jnp
from jax import lax
from jax.experimental import pallas as pl
from jax.experimental.pallas import tpu as pltpu
from jax.experimental.pallas import tpu_sc as plsc

VOCAB = 100000
ROW = 256
NFLAT = 4096 * 200
NC, NS, NW, L = 2, 16, 32, 16
VPAD = 102400
SCAN_W = VPAD // NW
SEQ_W = NFLAT // NW
CHUNK = 128
K1_CHUNKS = SEQ_W // CHUNK
K3_CHUNKS = 25

_mesh = plsc.VectorSubcoreMesh(core_axis_name="c", subcore_axis_name="s")
_sc_params = pltpu.CompilerParams(needs_layout_passes=False)


def _wid():
    return lax.axis_index("c") * NS + lax.axis_index("s")


K1_RING = 8


def _k1_body(seq_hbm, present0_hbm, present1_hbm, *refs):
    bufs = refs[:K1_RING]
    ones_v = refs[K1_RING]
    z_v = refs[K1_RING + 1]
    bitmap_sp = refs[K1_RING + 2]
    slo = refs[K1_RING + 3:K1_RING + 3 + K1_RING]
    ssc = refs[K1_RING + 3 + K1_RING:]
    c = lax.axis_index("c")
    s = lax.axis_index("s")
    wid = c * NS + s

    def zero_z(i, _):
        z_v[pl.ds(i * L, L)] = jnp.zeros((L,), jnp.int32)
        return 0

    lax.fori_loop(0, (VPAD // NS) // L, zero_z, 0)
    pltpu.sync_copy(z_v, bitmap_sp.at[pl.ds(s * (VPAD // NS), VPAD // NS)])
    plsc.subcore_barrier()

    def fill_ones(i, _):
        ones_v[pl.ds(i * L, L)] = jnp.ones((L,), jnp.int32)
        return 0

    lax.fori_loop(0, CHUNK // L, fill_ones, 0)

    base = wid * SEQ_W

    def start_load(j, k):
        pltpu.async_copy(seq_hbm.at[pl.ds(base + j * CHUNK, CHUNK)], bufs[k], slo[k])

    def wait_load(k):
        pltpu.make_async_copy(seq_hbm.at[pl.ds(0, CHUNK)], bufs[k], slo[k]).wait()

    def drain_sc(k):
        pltpu.make_async_copy(seq_hbm.at[pl.ds(0, CHUNK)], bufs[k], ssc[k]).wait()

    for k in range(K1_RING):
        start_load(k, k)

    def round_(r, _):
        for k in range(K1_RING):
            j = K1_RING * r + k
            wait_load(k)
            pltpu.async_copy(ones_v, bitmap_sp.at[bufs[k]], ssc[k])

            @pl.when(j + K1_RING < K1_CHUNKS)
            def _():
                drain_sc(k)
                start_load(j + K1_RING, k)
        return 0

    lax.fori_loop(0, K1_CHUNKS // K1_RING, round_, 0)
    for k in range(K1_RING):
        drain_sc(k)
    plsc.subcore_barrier()
    sl = pl.ds(s * (VPAD // NS), VPAD // NS)

    @pl.when(c == 0)
    def _():
        pltpu.sync_copy(bitmap_sp.at[sl], present0_hbm.at[sl])

    @pl.when(c == 1)
    def _():
        pltpu.sync_copy(bitmap_sp.at[sl], present1_hbm.at[sl])


_k1 = functools.partial(
    pl.kernel,
    out_type=(
        jax.ShapeDtypeStruct((VPAD,), jnp.int32),
        jax.ShapeDtypeStruct((VPAD,), jnp.int32),
    ),
    mesh=_mesh,
    compiler_params=_sc_params,
    scratch_types=(
        [pltpu.VMEM((CHUNK,), jnp.int32)] * K1_RING
        + [
            pltpu.VMEM((CHUNK,), jnp.int32),
            pltpu.VMEM((VPAD // NS,), jnp.int32),
            pltpu.VMEM_SHARED((VPAD,), jnp.int32),
        ]
        + [pltpu.SemaphoreType.DMA] * (2 * K1_RING)
    ),
)(_k1_body)


W_WIN = 3200


def _range_start(u):
    raw = lax.div(jnp.int32(3125) * u, jnp.int32(8)) * 8
    return jnp.where(
        u >= NW, jnp.int32(VOCAB), jnp.minimum(raw, jnp.int32(VOCAB - W_WIN))
    )


def _k2a_body(present0_hbm, present1_hbm, wsums_hbm, p0_v, p1_v, s_v):
    wid = _wid()
    b = pl.multiple_of(_range_start(wid), 8)
    n = _range_start(wid + 1) - b
    sl = pl.ds(b, W_WIN)
    pltpu.sync_copy(present0_hbm.at[sl], p0_v)
    pltpu.sync_copy(present1_hbm.at[sl], p1_v)
    iota = lax.iota(jnp.int32, L)

    def body(k, acc):
        p = p0_v[pl.ds(k * L, L)] + p1_v[pl.ds(k * L, L)]
        valid = (k * L + iota) < n
        return acc + jnp.where(valid & (p > 0), 1, 0).astype(jnp.int32)

    acc = lax.fori_loop(0, W_WIN // L, body, jnp.zeros((L,), jnp.int32))
    total = jnp.sum(acc)
    s_v[...] = jnp.full((L,), total, jnp.int32)
    pltpu.sync_copy(s_v, wsums_hbm.at[wid])


_k2a = functools.partial(
    pl.kernel,
    out_type=jax.ShapeDtypeStruct((NW, L), jnp.int32),
    mesh=_mesh,
    compiler_params=_sc_params,
    scratch_types=[
        pltpu.VMEM((W_WIN,), jnp.int32),
        pltpu.VMEM((W_WIN,), jnp.int32),
        pltpu.VMEM((L,), jnp.int32),
    ],
)(_k2a_body)


def _k3_body(table_hbm, present0_hbm, present1_hbm, wsums_hbm, out_hbm, nfill_hbm,
             rows0, rows1, idx0, idx1, p0_v, p1_v, w_v, nf_v,
             sg0, sg1, ss0, ss1):
    wid = _wid()
    rows = (rows0, rows1)
    idx = (idx0, idx1)
    sg = (sg0, sg1)
    ss = (ss0, ss1)
    b = pl.multiple_of(_range_start(wid), 8)

    pltpu.sync_copy(wsums_hbm, w_v)

    def sums(r, carry):
        total, offset = carry
        sr = jnp.max(w_v[r])
        return total + sr, offset + jnp.where(r < wid, sr, jnp.int32(0))

    total, offset = lax.fori_loop(0, NW, sums, (jnp.int32(0), jnp.int32(0)))
    n_fill = jnp.int32(VOCAB) - total

    @pl.when(wid == 0)
    def _():
        nf_v[...] = jnp.full((L,), n_fill, jnp.int32)
        pltpu.sync_copy(nf_v, nfill_hbm)

    sl = pl.ds(b, W_WIN)
    pltpu.sync_copy(present0_hbm.at[sl], p0_v)
    pltpu.sync_copy(present1_hbm.at[sl], p1_v)

    def scan_chunk(jj, carry, buf):
        for i in range(CHUNK // L):
            off = jj * CHUNK + i * L
            p = jnp.where(p0_v[pl.ds(off, L)] + p1_v[pl.ds(off, L)] > 0, 1, 0)
            p = p.astype(jnp.int32)
            incl = plsc.cumsum(p)
            buf[pl.ds(i * L, L)] = jnp.where(p > 0, n_fill + carry + (incl - p), 0)
            carry = carry + jnp.sum(p)
        return carry

    def start_gather(j, p):
        base = pl.multiple_of(b + j * CHUNK, 8)
        pltpu.async_copy(table_hbm.at[pl.ds(base, CHUNK)], rows[p], sg[p])

    def wait_gather(p):
        pltpu.make_async_copy(table_hbm.at[pl.ds(0, CHUNK)], rows[p], sg[p]).wait()

    def wait_scatter(p):
        pltpu.make_async_copy(table_hbm.at[pl.ds(0, CHUNK)], rows[p], ss[p]).wait()

    carry0 = scan_chunk(0, offset, idx0)
    start_gather(0, 0)

    def iter_t(t, carry):
        for phase in range(2):
            j = 2 * t + phase

            @pl.when(j >= 1)
            def _():
                wait_scatter(1 - phase)

            start_gather(j + 1, 1 - phase)
            carry = scan_chunk(j + 1, carry, idx[1 - phase])
            wait_gather(phase)
            pltpu.async_copy(rows[phase], out_hbm.at[idx[phase]], ss[phase])
        return carry

    lax.fori_loop(0, (K3_CHUNKS - 1) // 2, iter_t, carry0)
    wait_gather(0)
    pltpu.async_copy(rows[0], out_hbm.at[idx[0]], ss[0])
    wait_scatter(1)
    wait_scatter(0)


_k3 = functools.partial(
    pl.kernel,
    out_type=(
        jax.ShapeDtypeStruct((VOCAB, ROW), jnp.float32),
        jax.ShapeDtypeStruct((L,), jnp.int32),
    ),
    mesh=_mesh,
    compiler_params=_sc_params,
    scratch_types=[
        pltpu.VMEM((CHUNK, ROW), jnp.float32),
        pltpu.VMEM((CHUNK, ROW), jnp.float32),
        pltpu.VMEM((CHUNK,), jnp.int32),
        pltpu.VMEM((CHUNK,), jnp.int32),
        pltpu.VMEM((W_WIN,), jnp.int32),
        pltpu.VMEM((W_WIN,), jnp.int32),
        pltpu.VMEM((NW, L), jnp.int32),
        pltpu.VMEM((L,), jnp.int32),
        pltpu.SemaphoreType.DMA,
        pltpu.SemaphoreType.DMA,
        pltpu.SemaphoreType.DMA,
        pltpu.SemaphoreType.DMA,
    ],
)(_k3_body)


def _k4_body(nfill_ref, out_in_ref, out_ref, z_v, sem):
    del out_in_ref
    z_v[...] = jnp.zeros_like(z_v)
    n = nfill_ref[0]
    nb = n // 8

    def blk(i, _):
        cp = pltpu.make_async_copy(z_v, out_ref.at[pl.ds(i * 8, 8)], sem)
        cp.start()
        cp.wait()
        return 0

    lax.fori_loop(0, nb, blk, 0)

    def row(i, _):
        cp = pltpu.make_async_copy(
            z_v.at[pl.ds(0, 1)], out_ref.at[pl.ds(nb * 8 + i, 1)], sem
        )
        cp.start()
        cp.wait()
        return 0

    lax.fori_loop(0, n - nb * 8, row, 0)


_k4 = pl.pallas_call(
    _k4_body,
    out_shape=jax.ShapeDtypeStruct((VOCAB, ROW), jnp.float32),
    in_specs=[
        pl.BlockSpec(memory_space=pltpu.SMEM),
        pl.BlockSpec(memory_space=pl.ANY),
    ],
    out_specs=pl.BlockSpec(memory_space=pl.ANY),
    scratch_shapes=[pltpu.VMEM((8, ROW), jnp.float32), pltpu.SemaphoreType.DMA],
    input_output_aliases={1: 0},
)


@jax.jit
def kernel(sequence, node_table):
    table2 = node_table.reshape(VOCAB, ROW)
    present0, present1 = _k1(sequence.reshape(-1))
    wsums = _k2a(present0, present1)
    out, nfill = _k3(table2, present0, present1, wsums)
    out = _k4(nfill[:1], out)
    return out.reshape(VOCAB, 8, 32)


# speedup vs baseline: 46.0754x; 1.0226x over previous
import functools

import jax
import jax.numpy as jnp
from jax import lax
from jax.experimental import pallas as pl
from jax.experimental.pallas import tpu as pltpu
from jax.experimental.pallas import tpu_sc as plsc

VOCAB = 100000
ROW = 256
NFLAT = 4096 * 200
NC, NS, NW, L = 2, 16, 32, 16
VPAD = 102400
SCAN_W = VPAD // NW
SEQ_W = NFLAT // NW
CHUNK = 128
K1_CHUNKS = SEQ_W // CHUNK
K3_CHUNKS = 25

_mesh = plsc.VectorSubcoreMesh(core_axis_name="c", subcore_axis_name="s")
_sc_params = pltpu.CompilerParams(needs_layout_passes=False)


def _wid():
    return lax.axis_index("c") * NS + lax.axis_index("s")


K1_RING = 8


def _k1_body(seq_hbm, present0_hbm, present1_hbm, *refs):
    bufs = refs[:K1_RING]
    ones_v = refs[K1_RING]
    z_v = refs[K1_RING + 1]
    bitmap_sp = refs[K1_RING + 2]
    slo = refs[K1_RING + 3:K1_RING + 3 + K1_RING]
    ssc = refs[K1_RING + 3 + K1_RING:]
    c = lax.axis_index("c")
    s = lax.axis_index("s")
    wid = c * NS + s

    def zero_z(i, _):
        z_v[pl.ds(i * L, L)] = jnp.zeros((L,), jnp.int32)
        return 0

    lax.fori_loop(0, (VPAD // NS) // L, zero_z, 0)
    pltpu.sync_copy(z_v, bitmap_sp.at[pl.ds(s * (VPAD // NS), VPAD // NS)])
    plsc.subcore_barrier()

    def fill_ones(i, _):
        ones_v[pl.ds(i * L, L)] = jnp.ones((L,), jnp.int32)
        return 0

    lax.fori_loop(0, CHUNK // L, fill_ones, 0)

    base = wid * SEQ_W

    def start_load(j, k):
        pltpu.async_copy(seq_hbm.at[pl.ds(base + j * CHUNK, CHUNK)], bufs[k], slo[k])

    def wait_load(k):
        pltpu.make_async_copy(seq_hbm.at[pl.ds(0, CHUNK)], bufs[k], slo[k]).wait()

    def drain_sc(k):
        pltpu.make_async_copy(seq_hbm.at[pl.ds(0, CHUNK)], bufs[k], ssc[k]).wait()

    for k in range(K1_RING):
        start_load(k, k)

    def round_(r, _):
        for k in range(K1_RING):
            j = K1_RING * r + k
            wait_load(k)
            pltpu.async_copy(ones_v, bitmap_sp.at[bufs[k]], ssc[k])

            @pl.when(j + K1_RING < K1_CHUNKS)
            def _():
                drain_sc(k)
                start_load(j + K1_RING, k)
        return 0

    lax.fori_loop(0, K1_CHUNKS // K1_RING, round_, 0)
    for k in range(K1_RING):
        drain_sc(k)
    plsc.subcore_barrier()
    sl = pl.ds(s * (VPAD // NS), VPAD // NS)

    @pl.when(c == 0)
    def _():
        pltpu.sync_copy(bitmap_sp.at[sl], present0_hbm.at[sl])

    @pl.when(c == 1)
    def _():
        pltpu.sync_copy(bitmap_sp.at[sl], present1_hbm.at[sl])


_k1 = functools.partial(
    pl.kernel,
    out_type=(
        jax.ShapeDtypeStruct((VPAD,), jnp.int32),
        jax.ShapeDtypeStruct((VPAD,), jnp.int32),
    ),
    mesh=_mesh,
    compiler_params=_sc_params,
    scratch_types=(
        [pltpu.VMEM((CHUNK,), jnp.int32)] * K1_RING
        + [
            pltpu.VMEM((CHUNK,), jnp.int32),
            pltpu.VMEM((VPAD // NS,), jnp.int32),
            pltpu.VMEM_SHARED((VPAD,), jnp.int32),
        ]
        + [pltpu.SemaphoreType.DMA] * (2 * K1_RING)
    ),
)(_k1_body)


W_WIN = 3200


def _range_start(u):
    raw = lax.div(jnp.int32(3125) * u, jnp.int32(8)) * 8
    return jnp.where(
        u >= NW, jnp.int32(VOCAB), jnp.minimum(raw, jnp.int32(VOCAB - W_WIN))
    )


def _k2a_body(present0_hbm, present1_hbm, wsums_hbm, p0_v, p1_v, s_v):
    wid = _wid()
    b = pl.multiple_of(_range_start(wid), 8)
    n = _range_start(wid + 1) - b
    sl = pl.ds(b, W_WIN)
    pltpu.sync_copy(present0_hbm.at[sl], p0_v)
    pltpu.sync_copy(present1_hbm.at[sl], p1_v)
    iota = lax.iota(jnp.int32, L)

    def body(k, acc):
        p = p0_v[pl.ds(k * L, L)] + p1_v[pl.ds(k * L, L)]
        valid = (k * L + iota) < n
        return acc + jnp.where(valid & (p > 0), 1, 0).astype(jnp.int32)

    acc = lax.fori_loop(0, W_WIN // L, body, jnp.zeros((L,), jnp.int32))
    total = jnp.sum(acc)
    s_v[...] = jnp.full((L,), total, jnp.int32)
    pltpu.sync_copy(s_v, wsums_hbm.at[wid])


_k2a = functools.partial(
    pl.kernel,
    out_type=jax.ShapeDtypeStruct((NW, L), jnp.int32),
    mesh=_mesh,
    compiler_params=_sc_params,
    scratch_types=[
        pltpu.VMEM((W_WIN,), jnp.int32),
        pltpu.VMEM((W_WIN,), jnp.int32),
        pltpu.VMEM((L,), jnp.int32),
    ],
)(_k2a_body)


def _k3_body(table_hbm, present0_hbm, present1_hbm, wsums_hbm, out_hbm, nfill_hbm,
             rows0, rows1, rows2, idx0, idx1, idx2, p0_v, p1_v, w_v, nf_v,
             sg0, sg1, sg2, ss0, ss1, ss2):
    wid = _wid()
    rows = (rows0, rows1, rows2)
    idx = (idx0, idx1, idx2)
    sg = (sg0, sg1, sg2)
    ss = (ss0, ss1, ss2)
    b = pl.multiple_of(_range_start(wid), 8)

    pltpu.sync_copy(wsums_hbm, w_v)

    def sums(r, carry):
        total, offset = carry
        sr = jnp.max(w_v[r])
        return total + sr, offset + jnp.where(r < wid, sr, jnp.int32(0))

    total, offset = lax.fori_loop(0, NW, sums, (jnp.int32(0), jnp.int32(0)))
    n_fill = jnp.int32(VOCAB) - total

    @pl.when(wid == 0)
    def _():
        nf_v[...] = jnp.full((L,), n_fill, jnp.int32)
        pltpu.sync_copy(nf_v, nfill_hbm)

    sl = pl.ds(b, W_WIN)
    pltpu.sync_copy(present0_hbm.at[sl], p0_v)
    pltpu.sync_copy(present1_hbm.at[sl], p1_v)

    def scan_chunk(jj, carry, buf):
        for i in range(CHUNK // L):
            off = jj * CHUNK + i * L
            p = jnp.where(p0_v[pl.ds(off, L)] + p1_v[pl.ds(off, L)] > 0, 1, 0)
            p = p.astype(jnp.int32)
            incl = plsc.cumsum(p)
            buf[pl.ds(i * L, L)] = jnp.where(p > 0, n_fill + carry + (incl - p), 0)
            carry = carry + jnp.sum(p)
        return carry

    def start_gather(j, p):
        base = pl.multiple_of(b + j * CHUNK, 8)
        pltpu.async_copy(table_hbm.at[pl.ds(base, CHUNK)], rows[p], sg[p])

    def wait_gather(p):
        pltpu.make_async_copy(table_hbm.at[pl.ds(0, CHUNK)], rows[p], sg[p]).wait()

    def wait_scatter(p):
        pltpu.make_async_copy(table_hbm.at[pl.ds(0, CHUNK)], rows[p], ss[p]).wait()

    carry0 = scan_chunk(0, offset, idx0)
    start_gather(0, 0)

    def iter_t(t, carry):
        for phase in range(3):
            j = 3 * t + phase
            nb = (phase + 1) % 3

            @pl.when(j >= 2)
            def _():
                wait_scatter(nb)

            start_gather(j + 1, nb)
            carry = scan_chunk(j + 1, carry, idx[nb])
            wait_gather(phase)
            pltpu.async_copy(rows[phase], out_hbm.at[idx[phase]], ss[phase])
        return carry

    lax.fori_loop(0, (K3_CHUNKS - 1) // 3, iter_t, carry0)
    wait_gather(0)
    pltpu.async_copy(rows[0], out_hbm.at[idx[0]], ss[0])
    wait_scatter(1)
    wait_scatter(2)
    wait_scatter(0)


_k3 = functools.partial(
    pl.kernel,
    out_type=(
        jax.ShapeDtypeStruct((VOCAB, ROW), jnp.float32),
        jax.ShapeDtypeStruct((L,), jnp.int32),
    ),
    mesh=_mesh,
    compiler_params=_sc_params,
    scratch_types=[
        pltpu.VMEM((CHUNK, ROW), jnp.float32),
        pltpu.VMEM((CHUNK, ROW), jnp.float32),
        pltpu.VMEM((CHUNK, ROW), jnp.float32),
        pltpu.VMEM((CHUNK,), jnp.int32),
        pltpu.VMEM((CHUNK,), jnp.int32),
        pltpu.VMEM((CHUNK,), jnp.int32),
        pltpu.VMEM((W_WIN,), jnp.int32),
        pltpu.VMEM((W_WIN,), jnp.int32),
        pltpu.VMEM((NW, L), jnp.int32),
        pltpu.VMEM((L,), jnp.int32),
        pltpu.SemaphoreType.DMA,
        pltpu.SemaphoreType.DMA,
        pltpu.SemaphoreType.DMA,
        pltpu.SemaphoreType.DMA,
        pltpu.SemaphoreType.DMA,
        pltpu.SemaphoreType.DMA,
    ],
)(_k3_body)


def _k4_body(nfill_ref, out_in_ref, out_ref, z_v, sem):
    del out_in_ref
    z_v[...] = jnp.zeros_like(z_v)
    n = nfill_ref[0]
    nb = n // 8

    def blk(i, _):
        cp = pltpu.make_async_copy(z_v, out_ref.at[pl.ds(i * 8, 8)], sem)
        cp.start()
        cp.wait()
        return 0

    lax.fori_loop(0, nb, blk, 0)

    def row(i, _):
        cp = pltpu.make_async_copy(
            z_v.at[pl.ds(0, 1)], out_ref.at[pl.ds(nb * 8 + i, 1)], sem
        )
        cp.start()
        cp.wait()
        return 0

    lax.fori_loop(0, n - nb * 8, row, 0)


_k4 = pl.pallas_call(
    _k4_body,
    out_shape=jax.ShapeDtypeStruct((VOCAB, ROW), jnp.float32),
    in_specs=[
        pl.BlockSpec(memory_space=pltpu.SMEM),
        pl.BlockSpec(memory_space=pl.ANY),
    ],
    out_specs=pl.BlockSpec(memory_space=pl.ANY),
    scratch_shapes=[pltpu.VMEM((8, ROW), jnp.float32), pltpu.SemaphoreType.DMA],
    input_output_aliases={1: 0},
)


@jax.jit
def kernel(sequence, node_table):
    table2 = node_table.reshape(VOCAB, ROW)
    present0, present1 = _k1(sequence.T.reshape(-1))
    wsums = _k2a(present0, present1)
    out, nfill = _k3(table2, present0, present1, wsums)
    out = _k4(nfill[:1], out)
    return out.reshape(VOCAB, 8, 32)


# speedup vs baseline: 49.1820x; 1.0674x over previous
import functools

import jax
import jax.numpy as jnp
from jax import lax
from jax.experimental import pallas as pl
from jax.experimental.pallas import tpu as pltpu
from jax.experimental.pallas import tpu_sc as plsc

VOCAB = 100000
ROW = 256
NFLAT = 4096 * 200
NC, NS, NW, L = 2, 16, 32, 16
VPAD = 102400
SCAN_W = VPAD // NW
SEQ_W = NFLAT // NW
CHUNK = 128
K1_CHUNKS = SEQ_W // CHUNK
K3_CHUNKS = 25

_mesh = plsc.VectorSubcoreMesh(core_axis_name="c", subcore_axis_name="s")
_sc_params = pltpu.CompilerParams(needs_layout_passes=False)


def _wid():
    return lax.axis_index("c") * NS + lax.axis_index("s")


K1_RING = 8


def _k1_body(seq_hbm, present0_hbm, present1_hbm, *refs):
    bufs = refs[:K1_RING]
    ones_v = refs[K1_RING]
    z_v = refs[K1_RING + 1]
    bitmap_sp = refs[K1_RING + 2]
    slo = refs[K1_RING + 3:K1_RING + 3 + K1_RING]
    ssc = refs[K1_RING + 3 + K1_RING:]
    c = lax.axis_index("c")
    s = lax.axis_index("s")
    wid = c * NS + s

    def zero_z(i, _):
        z_v[pl.ds(i * L, L)] = jnp.zeros((L,), jnp.int32)
        return 0

    lax.fori_loop(0, (VPAD // NS) // L, zero_z, 0)
    pltpu.sync_copy(z_v, bitmap_sp.at[pl.ds(s * (VPAD // NS), VPAD // NS)])
    plsc.subcore_barrier()

    def fill_ones(i, _):
        ones_v[pl.ds(i * L, L)] = jnp.ones((L,), jnp.int32)
        return 0

    lax.fori_loop(0, CHUNK // L, fill_ones, 0)

    base = wid * SEQ_W

    def start_load(j, k):
        pltpu.async_copy(seq_hbm.at[pl.ds(base + j * CHUNK, CHUNK)], bufs[k], slo[k])

    def wait_load(k):
        pltpu.make_async_copy(seq_hbm.at[pl.ds(0, CHUNK)], bufs[k], slo[k]).wait()

    def drain_sc(k):
        pltpu.make_async_copy(seq_hbm.at[pl.ds(0, CHUNK)], bufs[k], ssc[k]).wait()

    for k in range(K1_RING):
        start_load(k, k)

    def round_(r, _):
        for k in range(K1_RING):
            j = K1_RING * r + k
            wait_load(k)
            pltpu.async_copy(ones_v, bitmap_sp.at[bufs[k]], ssc[k])

            @pl.when(j + K1_RING < K1_CHUNKS)
            def _():
                drain_sc(k)
                start_load(j + K1_RING, k)
        return 0

    lax.fori_loop(0, K1_CHUNKS // K1_RING, round_, 0)
    for k in range(K1_RING):
        drain_sc(k)
    plsc.subcore_barrier()
    sl = pl.ds(s * (VPAD // NS), VPAD // NS)

    @pl.when(c == 0)
    def _():
        pltpu.sync_copy(bitmap_sp.at[sl], present0_hbm.at[sl])

    @pl.when(c == 1)
    def _():
        pltpu.sync_copy(bitmap_sp.at[sl], present1_hbm.at[sl])


_k1 = functools.partial(
    pl.kernel,
    out_type=(
        jax.ShapeDtypeStruct((VPAD,), jnp.int32),
        jax.ShapeDtypeStruct((VPAD,), jnp.int32),
    ),
    mesh=_mesh,
    compiler_params=_sc_params,
    scratch_types=(
        [pltpu.VMEM((CHUNK,), jnp.int32)] * K1_RING
        + [
            pltpu.VMEM((CHUNK,), jnp.int32),
            pltpu.VMEM((VPAD // NS,), jnp.int32),
            pltpu.VMEM_SHARED((VPAD,), jnp.int32),
        ]
        + [pltpu.SemaphoreType.DMA] * (2 * K1_RING)
    ),
)(_k1_body)


W_WIN = 3200


def _range_start(u):
    raw = lax.div(jnp.int32(3125) * u, jnp.int32(8)) * 8
    return jnp.where(
        u >= NW, jnp.int32(VOCAB), jnp.minimum(raw, jnp.int32(VOCAB - W_WIN))
    )


def _k2a_body(present0_hbm, present1_hbm, table_hbm, wsums_hbm, p0_v, p1_v, s_v):
    del table_hbm
    wid = _wid()
    b = pl.multiple_of(_range_start(wid), 8)
    n = _range_start(wid + 1) - b
    sl = pl.ds(b, W_WIN)
    pltpu.sync_copy(present0_hbm.at[sl], p0_v)
    pltpu.sync_copy(present1_hbm.at[sl], p1_v)
    iota = lax.iota(jnp.int32, L)

    def body(k, acc):
        p = p0_v[pl.ds(k * L, L)] + p1_v[pl.ds(k * L, L)]
        valid = (k * L + iota) < n
        return acc + jnp.where(valid & (p > 0), 1, 0).astype(jnp.int32)

    acc = lax.fori_loop(0, W_WIN // L, body, jnp.zeros((L,), jnp.int32))
    total = jnp.sum(acc)
    s_v[...] = jnp.full((L,), total, jnp.int32)
    pltpu.sync_copy(s_v, wsums_hbm.at[wid])


_k2a = functools.partial(
    pl.kernel,
    out_type=jax.ShapeDtypeStruct((NW, L), jnp.int32),
    mesh=_mesh,
    compiler_params=_sc_params,
    scratch_types=[
        pltpu.VMEM((W_WIN,), jnp.int32),
        pltpu.VMEM((W_WIN,), jnp.int32),
        pltpu.VMEM((L,), jnp.int32),
    ],
)(_k2a_body)


def _k3_body(table_hbm, present0_hbm, present1_hbm, wsums_hbm, out_hbm, nfill_hbm,
             rows0, rows1, rows2, idx0, idx1, idx2, p0_v, p1_v, w_v, nf_v,
             sg0, sg1, sg2, ss0, ss1, ss2):
    wid = _wid()
    rows = (rows0, rows1, rows2)
    idx = (idx0, idx1, idx2)
    sg = (sg0, sg1, sg2)
    ss = (ss0, ss1, ss2)
    b = pl.multiple_of(_range_start(wid), 8)

    pltpu.sync_copy(wsums_hbm, w_v)

    def sums(r, carry):
        total, offset = carry
        sr = jnp.max(w_v[r])
        return total + sr, offset + jnp.where(r < wid, sr, jnp.int32(0))

    total, offset = lax.fori_loop(0, NW, sums, (jnp.int32(0), jnp.int32(0)))
    n_fill = jnp.int32(VOCAB) - total

    @pl.when(wid == 0)
    def _():
        nf_v[...] = jnp.full((L,), n_fill, jnp.int32)
        pltpu.sync_copy(nf_v, nfill_hbm)

    sl = pl.ds(b, W_WIN)
    pltpu.sync_copy(present0_hbm.at[sl], p0_v)
    pltpu.sync_copy(present1_hbm.at[sl], p1_v)

    def scan_chunk(jj, carry, buf):
        for i in range(CHUNK // L):
            off = jj * CHUNK + i * L
            p = jnp.where(p0_v[pl.ds(off, L)] + p1_v[pl.ds(off, L)] > 0, 1, 0)
            p = p.astype(jnp.int32)
            incl = plsc.cumsum(p)
            buf[pl.ds(i * L, L)] = jnp.where(p > 0, n_fill + carry + (incl - p), 0)
            carry = carry + jnp.sum(p)
        return carry

    def start_gather(j, p):
        base = pl.multiple_of(b + j * CHUNK, 8)
        pltpu.async_copy(table_hbm.at[pl.ds(base, CHUNK)], rows[p], sg[p])

    def wait_gather(p):
        pltpu.make_async_copy(table_hbm.at[pl.ds(0, CHUNK)], rows[p], sg[p]).wait()

    def wait_scatter(p):
        pltpu.make_async_copy(table_hbm.at[pl.ds(0, CHUNK)], rows[p], ss[p]).wait()

    carry0 = scan_chunk(0, offset, idx0)
    start_gather(0, 0)

    def iter_t(t, carry):
        for phase in range(3):
            j = 3 * t + phase
            nb = (phase + 1) % 3

            @pl.when(j >= 2)
            def _():
                wait_scatter(nb)

            start_gather(j + 1, nb)
            carry = scan_chunk(j + 1, carry, idx[nb])
            wait_gather(phase)
            pltpu.async_copy(rows[phase], out_hbm.at[idx[phase]], ss[phase])
        return carry

    lax.fori_loop(0, (K3_CHUNKS - 1) // 3, iter_t, carry0)
    wait_gather(0)
    pltpu.async_copy(rows[0], out_hbm.at[idx[0]], ss[0])
    wait_scatter(1)
    wait_scatter(2)
    wait_scatter(0)


_k3 = functools.partial(
    pl.kernel,
    out_type=(
        jax.ShapeDtypeStruct((VOCAB, ROW), jnp.float32),
        jax.ShapeDtypeStruct((L,), jnp.int32),
    ),
    mesh=_mesh,
    compiler_params=_sc_params,
    scratch_types=[
        pltpu.VMEM((CHUNK, ROW), jnp.float32),
        pltpu.VMEM((CHUNK, ROW), jnp.float32),
        pltpu.VMEM((CHUNK, ROW), jnp.float32),
        pltpu.VMEM((CHUNK,), jnp.int32),
        pltpu.VMEM((CHUNK,), jnp.int32),
        pltpu.VMEM((CHUNK,), jnp.int32),
        pltpu.VMEM((W_WIN,), jnp.int32),
        pltpu.VMEM((W_WIN,), jnp.int32),
        pltpu.VMEM((NW, L), jnp.int32),
        pltpu.VMEM((L,), jnp.int32),
        pltpu.SemaphoreType.DMA,
        pltpu.SemaphoreType.DMA,
        pltpu.SemaphoreType.DMA,
        pltpu.SemaphoreType.DMA,
        pltpu.SemaphoreType.DMA,
        pltpu.SemaphoreType.DMA,
    ],
)(_k3_body)


def _k4_body(nfill_ref, out_in_ref, out_ref, z_v, sem):
    del out_in_ref
    z_v[...] = jnp.zeros_like(z_v)
    n = nfill_ref[0]
    nb = n // 8

    def blk(i, _):
        cp = pltpu.make_async_copy(z_v, out_ref.at[pl.ds(i * 8, 8)], sem)
        cp.start()
        cp.wait()
        return 0

    lax.fori_loop(0, nb, blk, 0)

    def row(i, _):
        cp = pltpu.make_async_copy(
            z_v.at[pl.ds(0, 1)], out_ref.at[pl.ds(nb * 8 + i, 1)], sem
        )
        cp.start()
        cp.wait()
        return 0

    lax.fori_loop(0, n - nb * 8, row, 0)


_k4 = pl.pallas_call(
    _k4_body,
    out_shape=jax.ShapeDtypeStruct((VOCAB, ROW), jnp.float32),
    in_specs=[
        pl.BlockSpec(memory_space=pltpu.SMEM),
        pl.BlockSpec(memory_space=pl.ANY),
    ],
    out_specs=pl.BlockSpec(memory_space=pl.ANY),
    scratch_shapes=[pltpu.VMEM((8, ROW), jnp.float32), pltpu.SemaphoreType.DMA],
    input_output_aliases={1: 0},
)


@jax.jit
def kernel(sequence, node_table):
    table2 = node_table.reshape(VOCAB, ROW)
    present0, present1 = _k1(sequence.T.reshape(-1))
    wsums = _k2a(present0, present1, table2)
    out, nfill = _k3(table2, present0, present1, wsums)
    out = _k4(nfill[:1], out)
    return out.reshape(VOCAB, 8, 32)
